# Initial kernel scaffold; baseline (speedup 1.0000x reference)
import jax
import jax.numpy as jnp
from jax.experimental import pallas as pl


def kernel(x, table):
    raise NotImplementedError("write your pallas kernel here")


# speedup vs baseline: 4.7794x; 4.7794x over previous
import functools

import jax
import jax.numpy as jnp
from jax import lax
from jax.experimental import pallas as pl
from jax.experimental.pallas import tpu as pltpu
from jax.experimental.pallas import tpu_sc as plsc

B = 4096
L = 200
D = 64
V = 201
NW = 32
TOK = B * L
TPW = TOK // NW
CHUNK = 256
NBUF = 4
GROUP = CHUNK * NBUF
NIT = TPW // GROUP
IPG = 128
NG = CHUNK // IPG

_mesh = plsc.VectorSubcoreMesh(core_axis_name="c", subcore_axis_name="s")


@functools.partial(
    pl.kernel,
    mesh=_mesh,
    out_type=jax.ShapeDtypeStruct((TOK, D), jnp.float32),
    scratch_types=[
        pltpu.VMEM_SHARED((V, D), jnp.float32),
        pltpu.VMEM((NBUF, CHUNK), jnp.int32),
        pltpu.VMEM((NBUF, NG, IPG), jnp.int32),
        pltpu.VMEM((NBUF, CHUNK, D), jnp.float32),
        pltpu.SemaphoreType.DMA,
        [pltpu.SemaphoreType.DMA] * NBUF,
    ],
    compiler_params=pltpu.CompilerParams(use_tc_tiling_on_sc=False),
)
def _emb(x_hbm, table_hbm, out_hbm, tab_s, x_v, idx_v, rows_v, sem_g, sems_out):
    sid = lax.axis_index("s")
    wid = sid * 2 + lax.axis_index("c")
    wbase = wid * TPW
    lanes = lax.iota(jnp.int32, 16)

    @pl.when(sid == 0)
    def _():
        pltpu.sync_copy(table_hbm, tab_s)

    plsc.subcore_barrier()

    def fill(cbase, b):
        pltpu.sync_copy(x_hbm.at[pl.ds(cbase, CHUNK)], x_v.at[b])
        for j in range(CHUNK // 16):
            xv = x_v[b, pl.ds(j * 16, 16)]
            p = cbase + j * 16 + lanes
            col = lax.rem(p, L)
            pos = jnp.where(xv != 0, col + 1, 0)
            idx_v[b, j // 8, pl.ds((j % 8) * 16, 16)] = pos
        return [
            pltpu.async_copy(
                tab_s.at[idx_v.at[b, g]],
                rows_v.at[b, pl.ds(g * IPG, IPG)],
                sem_g,
            )
            for g in range(NG)
        ]

    def out_copy(cbase, b):
        return pltpu.make_async_copy(
            rows_v.at[b], out_hbm.at[pl.ds(cbase, CHUNK)], sems_out[b]
        )

    def body(i, carry):
        gbase = wbase + i * GROUP
        gathers = []
        for b in range(NBUF):
            @pl.when(i > 0)
            def _(b=b, gbase=gbase):
                out_copy(gbase - GROUP + b * CHUNK, b).wait()

            gathers.append(fill(gbase + b * CHUNK, b))
        for b in range(NBUF):
            for c in gathers[b]:
                c.wait()
            out_copy(gbase + b * CHUNK, b).start()
        return carry

    lax.fori_loop(0, NIT, body, 0)
    for b in range(NBUF):
        out_copy(wbase + (NIT - 1) * GROUP + b * CHUNK, b).wait()


def kernel(x, table):
    out = _emb(x.reshape(TOK), table)
    return out.reshape(B, L, D)


# speedup vs baseline: 9.2046x; 1.9259x over previous
import functools

import jax
import jax.numpy as jnp
from jax import lax
from jax.experimental import pallas as pl
from jax.experimental.pallas import tpu as pltpu
from jax.experimental.pallas import tpu_sc as plsc

B = 4096
L = 200
D = 64
V = 201
NW = 32
BPW = B // NW
LC = 4
NBUF = 2
NIT = L // (LC * NBUF)

_mesh = plsc.VectorSubcoreMesh(core_axis_name="c", subcore_axis_name="s")


@functools.partial(
    pl.kernel,
    mesh=_mesh,
    out_type=jax.ShapeDtypeStruct((L, D, B), jnp.float32),
    scratch_types=[
        pltpu.VMEM((V * D,), jnp.float32),
        pltpu.VMEM((NBUF, LC, BPW), jnp.int32),
        pltpu.VMEM((NBUF, LC, D, BPW), jnp.float32),
        [pltpu.SemaphoreType.DMA] * NBUF,
    ],
    compiler_params=pltpu.CompilerParams(needs_layout_passes=False),
)
def _emb(xt_hbm, tabf_hbm, out_hbm, tab_v, x_v, o_v, sems_out):
    wid = lax.axis_index("s") * 2 + lax.axis_index("c")
    bbase = wid * BPW
    pltpu.sync_copy(tabf_hbm, tab_v)
    zeros = jnp.zeros((16,), jnp.float32)
    lanes = lax.iota(jnp.int32, 16)

    def out_copy(l0, b):
        return pltpu.make_async_copy(
            o_v.at[b],
            out_hbm.at[pl.ds(l0, LC), :, pl.ds(bbase, BPW)],
            sems_out[b],
        )

    def fill(l0, b):
        pltpu.sync_copy(
            xt_hbm.at[pl.ds(l0, LC), pl.ds(bbase, BPW)], x_v.at[b]
        )
        for li in range(LC):
            base_idx = (l0 + li + 1) * D + lanes * 0
            masks = [
                x_v[b, li, pl.ds(g * 16, 16)] != 0 for g in range(BPW // 16)
            ]
            for d in range(D):
                trow = plsc.load_gather(tab_v, [base_idx + d])
                for g in range(BPW // 16):
                    o_v[b, li, d, pl.ds(g * 16, 16)] = jnp.where(
                        masks[g], trow, zeros
                    )

    def body(i, carry):
        base = i * (LC * NBUF)
        for b in range(NBUF):
            l0 = base + b * LC

            @pl.when(i > 0)
            def _(l0=l0, b=b):
                out_copy(l0 - LC * NBUF, b).wait()

            fill(l0, b)
            out_copy(l0, b).start()
        return carry

    lax.fori_loop(0, NIT, body, 0)
    for b in range(NBUF):
        out_copy((NIT - 1) * LC * NBUF + b * LC, b).wait()


def kernel(x, table):
    out = _emb(x.T, table.reshape(V * D))
    return out.transpose(2, 0, 1)


# speedup vs baseline: 17.8466x; 1.9389x over previous
import functools

import jax
import jax.numpy as jnp
from jax import lax
from jax.experimental import pallas as pl
from jax.experimental.pallas import tpu as pltpu
from jax.experimental.pallas import tpu_sc as plsc

B = 4096
L = 200
D = 64
V = 201
NW = 32
DB = 8
NUNITS = L * (D // DB)
UPW = NUNITS // NW
NBUF = 2
G16 = B // 16
UNROLL = 8

_mesh = plsc.VectorSubcoreMesh(core_axis_name="c", subcore_axis_name="s")


@functools.partial(
    pl.kernel,
    mesh=_mesh,
    out_type=jax.ShapeDtypeStruct((L, D, B), jnp.float32),
    scratch_types=[
        pltpu.VMEM((V * D,), jnp.float32),
        pltpu.VMEM((NBUF, B), jnp.int32),
        pltpu.VMEM((NBUF, DB, B), jnp.float32),
        [pltpu.SemaphoreType.DMA] * NBUF,
        [pltpu.SemaphoreType.DMA] * NBUF,
    ],
    compiler_params=pltpu.CompilerParams(needs_layout_passes=False),
)
def _emb(xt_hbm, tabf_hbm, out_hbm, tab_v, x_v, o_v, sems_x, sems_out):
    wid = lax.axis_index("s") * 2 + lax.axis_index("c")
    ubase = wid * UPW
    pltpu.sync_copy(tabf_hbm, tab_v)
    zeros = jnp.zeros((16,), jnp.float32)
    zlanes = lax.iota(jnp.int32, 16) * 0

    def unit_lb(u):
        return u // DB, (u % DB) * DB

    def x_copy(u, b):
        l, _ = unit_lb(u)
        return pltpu.make_async_copy(xt_hbm.at[l], x_v.at[b], sems_x[b])

    def out_copy(u, b):
        l, d0 = unit_lb(u)
        return pltpu.make_async_copy(
            o_v.at[b], out_hbm.at[l, pl.ds(d0, DB)], sems_out[b]
        )

    def compute(u, b):
        l, d0 = unit_lb(u)
        base = (l + 1) * D + d0
        trows = [
            plsc.load_gather(tab_v, [base + d + zlanes]) for d in range(DB)
        ]

        def gbody(g, carry):
            for gu in range(UNROLL):
                o = (g * UNROLL + gu) * 16
                m = x_v[b, pl.ds(o, 16)] != 0
                for d in range(DB):
                    o_v[b, d, pl.ds(o, 16)] = jnp.where(m, trows[d], zeros)
            return carry

        lax.fori_loop(0, G16 // UNROLL, gbody, 0)

    def body(i, carry):
        base = ubase + i * NBUF
        for b in range(NBUF):
            u = base + b

            @pl.when(i > 0)
            def _(u=u, b=b):
                out_copy(u - NBUF, b).wait()

            x_copy(u, b).start()
        for b in range(NBUF):
            u = base + b
            x_copy(u, b).wait()
            compute(u, b)
            out_copy(u, b).start()
        return carry

    lax.fori_loop(0, UPW // NBUF, body, 0)
    for b in range(NBUF):
        out_copy(ubase + UPW - NBUF + b, b).wait()


def kernel(x, table):
    out = _emb(x.T, table.reshape(V * D))
    return out.transpose(2, 0, 1)


# speedup vs baseline: 18.9227x; 1.0603x over previous
import functools

import jax
import jax.numpy as jnp
from jax import lax
from jax.experimental import pallas as pl
from jax.experimental.pallas import tpu as pltpu
from jax.experimental.pallas import tpu_sc as plsc

B = 4096
L = 200
D = 64
V = 201
NW = 32
DB = 8
BH = 2048
NBH = B // BH
UPL = (D // DB) * NBH
NUNITS = L * UPL
UPW = NUNITS // NW
NBUF = 4
G16 = BH // 16
UNROLL = 8

_mesh = plsc.VectorSubcoreMesh(core_axis_name="c", subcore_axis_name="s")


@functools.partial(
    pl.kernel,
    mesh=_mesh,
    out_type=jax.ShapeDtypeStruct((L, D, B), jnp.float32),
    scratch_types=[
        pltpu.VMEM((V * D,), jnp.float32),
        pltpu.VMEM((NBUF, BH), jnp.int32),
        pltpu.VMEM((NBUF, DB, BH), jnp.float32),
        [pltpu.SemaphoreType.DMA] * NBUF,
        [pltpu.SemaphoreType.DMA] * NBUF,
    ],
    compiler_params=pltpu.CompilerParams(needs_layout_passes=False),
)
def _emb(xt_hbm, tabf_hbm, out_hbm, tab_v, x_v, o_v, sems_x, sems_out):
    wid = lax.axis_index("s") * 2 + lax.axis_index("c")
    ubase = wid * UPW
    pltpu.sync_copy(tabf_hbm, tab_v)
    zeros = jnp.zeros((16,), jnp.float32)
    zlanes = lax.iota(jnp.int32, 16) * 0

    def unit_ldb(u):
        l = u // UPL
        r = u % UPL
        return l, (r // NBH) * DB, (r % NBH) * BH

    def x_copy(u, b):
        l, _, b0 = unit_ldb(u)
        return pltpu.make_async_copy(
            xt_hbm.at[l, pl.ds(b0, BH)], x_v.at[b], sems_x[b]
        )

    def out_copy(u, b):
        l, d0, b0 = unit_ldb(u)
        return pltpu.make_async_copy(
            o_v.at[b], out_hbm.at[l, pl.ds(d0, DB), pl.ds(b0, BH)], sems_out[b]
        )

    def compute(u, b):
        l, d0, _ = unit_ldb(u)
        base = (l + 1) * D + d0
        trows = [
            plsc.load_gather(tab_v, [base + d + zlanes]) for d in range(DB)
        ]

        def gbody(g, carry):
            for gu in range(UNROLL):
                o = (g * UNROLL + gu) * 16
                m = x_v[b, pl.ds(o, 16)] != 0
                for d in range(DB):
                    o_v[b, d, pl.ds(o, 16)] = jnp.where(m, trows[d], zeros)
            return carry

        lax.fori_loop(0, G16 // UNROLL, gbody, 0)

    def body(i, carry):
        base = ubase + i * NBUF
        for b in range(NBUF):
            u = base + b

            @pl.when(i > 0)
            def _(u=u, b=b):
                out_copy(u - NBUF, b).wait()

            x_copy(u, b).start()
        for b in range(NBUF):
            u = base + b
            x_copy(u, b).wait()
            compute(u, b)
            out_copy(u, b).start()
        return carry

    lax.fori_loop(0, UPW // NBUF, body, 0)
    for b in range(NBUF):
        out_copy(ubase + UPW - NBUF + b, b).wait()


def kernel(x, table):
    out = _emb(x.T, table.reshape(V * D))
    return out.transpose(2, 0, 1)


# speedup vs baseline: 20.6632x; 1.0920x over previous
import functools

import jax
import jax.numpy as jnp
from jax import lax
from jax.experimental import pallas as pl
from jax.experimental.pallas import tpu as pltpu
from jax.experimental.pallas import tpu_sc as plsc

B = 4096
L = 200
D = 64
V = 201
NW = 32
DB = 8
BH = 2048
NBH = B // BH
UPL = (D // DB) * NBH
NUNITS = L * UPL
UPW = NUNITS // NW
NBUF = 4
G16 = BH // 16
UNROLL = 8

_mesh = plsc.VectorSubcoreMesh(core_axis_name="c", subcore_axis_name="s")


@functools.partial(
    pl.kernel,
    mesh=_mesh,
    out_type=jax.ShapeDtypeStruct((L, D, B), jnp.float32),
    scratch_types=[
        pltpu.VMEM((V * D,), jnp.float32),
        pltpu.VMEM((NBUF, BH), jnp.int32),
        pltpu.VMEM((NBUF, DB, BH), jnp.float32),
        [pltpu.SemaphoreType.DMA] * NBUF,
        [pltpu.SemaphoreType.DMA] * NBUF,
    ],
    compiler_params=pltpu.CompilerParams(needs_layout_passes=False),
)
def _emb(xt_hbm, tabf_hbm, out_hbm, tab_v, x_v, o_v, sems_x, sems_out):
    wid = lax.axis_index("s") * 2 + lax.axis_index("c")
    ubase = wid * UPW
    pltpu.sync_copy(tabf_hbm, tab_v)
    zeros = jnp.zeros((16,), jnp.float32)
    zlanes = lax.iota(jnp.int32, 16) * 0

    def unit_ldb(u):
        l = u // UPL
        r = u % UPL
        return l, (r // NBH) * DB, (r % NBH) * BH

    def x_copy(u, b):
        l, _, b0 = unit_ldb(u)
        return pltpu.make_async_copy(
            xt_hbm.at[l, pl.ds(b0, BH)], x_v.at[b], sems_x[b]
        )

    def out_copy(u, b):
        l, d0, b0 = unit_ldb(u)
        return pltpu.make_async_copy(
            o_v.at[b], out_hbm.at[l, pl.ds(d0, DB), pl.ds(b0, BH)], sems_out[b]
        )

    def compute(u, b):
        l, d0, _ = unit_ldb(u)
        base = (l + 1) * D + d0
        trows = [
            plsc.load_gather(tab_v, [base + d + zlanes]) for d in range(DB)
        ]

        def gbody(g, carry):
            for gu in range(UNROLL):
                o = (g * UNROLL + gu) * 16
                for d in range(DB):
                    o_v[b, d, pl.ds(o, 16)] = trows[d]
            return carry

        lax.fori_loop(0, G16 // UNROLL, gbody, 0)

        SPAN = 256
        NSPAN = BH // SPAN

        def sbody(s, carry):
            o0 = s * SPAN
            acc = x_v[b, pl.ds(o0, 16)]
            for k in range(1, SPAN // 16):
                acc = jnp.minimum(acc, x_v[b, pl.ds(o0 + k * 16, 16)])
            amin = lax.reduce_min(acc, (0,))

            @pl.when(amin == 0)
            def _():
                def fbody(g, c2):
                    o = o0 + g * 16
                    m = x_v[b, pl.ds(o, 16)] != 0
                    for d in range(DB):
                        o_v[b, d, pl.ds(o, 16)] = jnp.where(
                            m, trows[d], zeros
                        )
                    return c2

                lax.fori_loop(0, SPAN // 16, fbody, 0)

            return carry

        lax.fori_loop(0, NSPAN, sbody, 0)

    def body(i, carry):
        base = ubase + i * NBUF
        for b in range(NBUF):
            u = base + b

            @pl.when(i > 0)
            def _(u=u, b=b):
                out_copy(u - NBUF, b).wait()

            x_copy(u, b).start()
        for b in range(NBUF):
            u = base + b
            x_copy(u, b).wait()
            compute(u, b)
            out_copy(u, b).start()
        return carry

    lax.fori_loop(0, UPW // NBUF, body, 0)
    for b in range(NBUF):
        out_copy(ubase + UPW - NBUF + b, b).wait()


def kernel(x, table):
    out = _emb(x.T, table.reshape(V * D))
    return out.transpose(2, 0, 1)


# speedup vs baseline: 21.5229x; 1.0416x over previous
import functools

import jax
import jax.numpy as jnp
from jax import lax
from jax.experimental import pallas as pl
from jax.experimental.pallas import tpu as pltpu
from jax.experimental.pallas import tpu_sc as plsc

B = 4096
L = 200
D = 64
V = 201
NW = 32
DB = 8
BH = 2048
NBH = B // BH
UPL = (D // DB) * NBH
NUNITS = L * UPL
UPW = NUNITS // NW
NBUF = 4
G16 = BH // 16
UNROLL = 8

_mesh = plsc.VectorSubcoreMesh(core_axis_name="c", subcore_axis_name="s")


@functools.partial(
    pl.kernel,
    mesh=_mesh,
    out_type=jax.ShapeDtypeStruct((L, D, B), jnp.float32),
    scratch_types=[
        pltpu.VMEM((V * D,), jnp.float32),
        pltpu.VMEM((NBUF, BH), jnp.int32),
        pltpu.VMEM((NBUF, DB, BH), jnp.float32),
        [pltpu.SemaphoreType.DMA] * NBUF,
        [pltpu.SemaphoreType.DMA] * NBUF,
    ],
    compiler_params=pltpu.CompilerParams(needs_layout_passes=False),
)
def _emb(xt_hbm, tabf_hbm, out_hbm, tab_v, x_v, o_v, sems_x, sems_out):
    wid = lax.axis_index("s") * 2 + lax.axis_index("c")
    ubase = wid * UPW
    pltpu.sync_copy(tabf_hbm, tab_v)
    zeros = jnp.zeros((16,), jnp.float32)
    zlanes = lax.iota(jnp.int32, 16) * 0

    def unit_ldb(u):
        l = u // UPL
        r = u % UPL
        return l, (r // NBH) * DB, (r % NBH) * BH

    def x_copy(u, b):
        l, _, b0 = unit_ldb(u)
        return pltpu.make_async_copy(
            xt_hbm.at[l, pl.ds(b0, BH)], x_v.at[b], sems_x[b]
        )

    def out_copy(u, b):
        l, d0, b0 = unit_ldb(u)
        return pltpu.make_async_copy(
            o_v.at[b], out_hbm.at[l, pl.ds(d0, DB), pl.ds(b0, BH)], sems_out[b]
        )

    def compute(u, b):
        l, d0, _ = unit_ldb(u)
        base = (l + 1) * D + d0
        trows = [
            plsc.load_gather(tab_v, [base + d + zlanes]) for d in range(DB)
        ]

        def gbody(g, carry):
            for gu in range(UNROLL):
                o = (g * UNROLL + gu) * 16
                for d in range(DB):
                    o_v[b, d, pl.ds(o, 16)] = trows[d]
            return carry

        lax.fori_loop(0, G16 // UNROLL, gbody, 0)

        accs = [x_v[b, pl.ds(a * 16, 16)] for a in range(4)]
        for k in range(4, G16):
            accs[k % 4] = jnp.minimum(
                accs[k % 4], x_v[b, pl.ds(k * 16, 16)]
            )
        acc = jnp.minimum(
            jnp.minimum(accs[0], accs[1]), jnp.minimum(accs[2], accs[3])
        )
        amin = lax.reduce_min(acc, (0,))

        @pl.when(amin == 0)
        def _():
            def fbody(g, c2):
                o = g * 16
                m = x_v[b, pl.ds(o, 16)] != 0
                for d in range(DB):
                    o_v[b, d, pl.ds(o, 16)] = jnp.where(m, trows[d], zeros)
                return c2

            lax.fori_loop(0, G16, fbody, 0)

    def body(i, carry):
        base = ubase + i * NBUF
        for b in range(NBUF):
            u = base + b

            @pl.when(i > 0)
            def _(u=u, b=b):
                out_copy(u - NBUF, b).wait()

            x_copy(u, b).start()
        for b in range(NBUF):
            u = base + b
            x_copy(u, b).wait()
            compute(u, b)
            out_copy(u, b).start()
        return carry

    lax.fori_loop(0, UPW // NBUF, body, 0)
    for b in range(NBUF):
        out_copy(ubase + UPW - NBUF + b, b).wait()


def kernel(x, table):
    out = _emb(x.T, table.reshape(V * D))
    return out.transpose(2, 0, 1)


# speedup vs baseline: 22.9867x; 1.0680x over previous
import functools

import jax
import jax.numpy as jnp
from jax import lax
from jax.experimental import pallas as pl
from jax.experimental.pallas import tpu as pltpu
from jax.experimental.pallas import tpu_sc as plsc

B = 4096
L = 200
D = 64
V = 201
NW = 32
DB = 8
BH = 2048
NBH = B // BH
UPL = (D // DB) * NBH
NUNITS = L * UPL
UPW = NUNITS // NW
NBUF = 4
G16 = BH // 16
UNROLL = 8

_mesh = plsc.VectorSubcoreMesh(core_axis_name="c", subcore_axis_name="s")


@functools.partial(
    pl.kernel,
    mesh=_mesh,
    out_type=jax.ShapeDtypeStruct((L, D, B), jnp.float32),
    scratch_types=[
        pltpu.VMEM((V * D,), jnp.float32),
        pltpu.VMEM((NBUF, BH), jnp.int32),
        pltpu.VMEM((NBUF, DB, BH), jnp.float32),
        [pltpu.SemaphoreType.DMA] * NBUF,
        [pltpu.SemaphoreType.DMA] * NBUF,
    ],
    compiler_params=pltpu.CompilerParams(needs_layout_passes=False),
)
def _emb(xt_hbm, tabf_hbm, out_hbm, tab_v, x_v, o_v, sems_x, sems_out):
    wid = lax.axis_index("s") * 2 + lax.axis_index("c")
    ubase = wid * UPW
    pltpu.sync_copy(tabf_hbm, tab_v)
    zeros = jnp.zeros((16,), jnp.float32)
    zlanes = lax.iota(jnp.int32, 16) * 0

    def unit_ldb(u):
        l = u // UPL
        r = u % UPL
        return l, (r // NBH) * DB, (r % NBH) * BH

    def x_copy(u, b):
        l, _, b0 = unit_ldb(u)
        return pltpu.make_async_copy(
            xt_hbm.at[l, pl.ds(b0, BH)], x_v.at[b], sems_x[b]
        )

    def out_copy(u, b):
        l, d0, b0 = unit_ldb(u)
        return pltpu.make_async_copy(
            o_v.at[b], out_hbm.at[l, pl.ds(d0, DB), pl.ds(b0, BH)], sems_out[b]
        )

    def compute(u, b):
        l, d0, _ = unit_ldb(u)
        base = (l + 1) * D + d0
        trows = [
            plsc.load_gather(tab_v, [base + d + zlanes]) for d in range(DB)
        ]

        big = jnp.full((16,), 2147483647, jnp.int32)

        def gbody(g, accs):
            accs = list(accs)
            for gu in range(UNROLL):
                o = (g * UNROLL + gu) * 16
                accs[gu % 4] = jnp.minimum(accs[gu % 4], x_v[b, pl.ds(o, 16)])
                for d in range(DB):
                    o_v[b, d, pl.ds(o, 16)] = trows[d]
            return tuple(accs)

        accs = lax.fori_loop(0, G16 // UNROLL, gbody, (big, big, big, big))

        acc = jnp.minimum(
            jnp.minimum(accs[0], accs[1]), jnp.minimum(accs[2], accs[3])
        )
        amin = lax.reduce_min(acc, (0,))

        @pl.when(amin == 0)
        def _():
            def fbody(g, c2):
                o = g * 16
                m = x_v[b, pl.ds(o, 16)] != 0
                for d in range(DB):
                    o_v[b, d, pl.ds(o, 16)] = jnp.where(m, trows[d], zeros)
                return c2

            lax.fori_loop(0, G16, fbody, 0)

    def body(i, carry):
        base = ubase + i * NBUF
        for b in range(NBUF):
            u = base + b

            @pl.when(i > 0)
            def _(u=u, b=b):
                out_copy(u - NBUF, b).wait()

            x_copy(u, b).start()
        for b in range(NBUF):
            u = base + b
            x_copy(u, b).wait()
            compute(u, b)
            out_copy(u, b).start()
        return carry

    lax.fori_loop(0, UPW // NBUF, body, 0)
    for b in range(NBUF):
        out_copy(ubase + UPW - NBUF + b, b).wait()


def kernel(x, table):
    out = _emb(x.T, table.reshape(V * D))
    return out.transpose(2, 0, 1)
